# baseline (speedup 1.0000x reference)
import jax
import jax.numpy as jnp
from jax import lax
from jax.experimental import pallas as pl
from jax.experimental.pallas import tpu as pltpu
from jax.experimental.pallas import tpu_sc as plsc

_VOCAB = 100000
_MAX_LEN = 2048
_EMBED_DIM = 128
_BATCH = 4
_B = _BATCH * _MAX_LEN
_NC = 2
_NS = 16
_NW = _NC * _NS
_BPW = _B // _NW
_H = _BPW // 2
_Q = _BPW // 4


def _embed_body(seq_hbm, tok_hbm, pos_hbm, out_hbm,
                idx_v, buf0, buf1, buf2, buf3, sh, sem_i,
                sem_p0, sem_p1, sem_p2, sem_p3,
                sem_g0, sem_g1, sem_g2, sem_g3,
                sem_s0, sem_s1, sem_s2, sem_s3):
    wid = lax.axis_index("s") * _NC + lax.axis_index("c")
    base = wid * _BPW
    b = base // _MAX_LEN
    col = lax.rem(base, _MAX_LEN)
    bufs = (buf0, buf1, buf2, buf3)
    sem_p = (sem_p0, sem_p1, sem_p2, sem_p3)
    sem_g = (sem_g0, sem_g1, sem_g2, sem_g3)
    sem_s = (sem_s0, sem_s1, sem_s2, sem_s3)

    icopy = pltpu.async_copy(seq_hbm.at[b, pl.ds(col, _BPW)], idx_v, sem_i)
    sid = lax.axis_index("s")
    cid = lax.axis_index("c")
    r = cid * 256 + (sid // 4) * 512 + lax.rem(sid, 4) * 64
    pltpu.sync_copy(pos_hbm.at[pl.ds(r, 64)], sh.at[pl.ds(sid * 64, 64)])
    plsc.subcore_barrier()
    k = col // 512
    ps = [pltpu.async_copy(sh.at[pl.ds(k * 256 + q * _Q, _Q)], bufs[q],
                           sem_p[q]) for q in range(4)]
    icopy.wait()
    gs = [None] * 4
    for q in range(4):
        ps[q].wait()
        gs[q] = pltpu.async_copy(
            tok_hbm.at[idx_v.at[pl.ds(q * _Q, _Q)]], bufs[q], sem_g[q],
            add=True)
    ss = [None] * 4
    for q in range(4):
        gs[q].wait()
        ss[q] = pltpu.async_copy(
            bufs[q], out_hbm.at[b, pl.ds(col + q * _Q, _Q)], sem_s[q])
    for q in range(4):
        ss[q].wait()


@jax.jit
def _embed(seq, tok_table, pos_table):
    mesh = plsc.VectorSubcoreMesh(core_axis_name="c", subcore_axis_name="s")
    f = pl.kernel(
        _embed_body,
        mesh=mesh,
        out_type=jax.ShapeDtypeStruct((_BATCH, _MAX_LEN, _EMBED_DIM),
                                      jnp.float32),
        scratch_types=[
            pltpu.VMEM((_BPW,), jnp.int32),
            pltpu.VMEM((_Q, _EMBED_DIM), jnp.float32),
            pltpu.VMEM((_Q, _EMBED_DIM), jnp.float32),
            pltpu.VMEM((_Q, _EMBED_DIM), jnp.float32),
            pltpu.VMEM((_Q, _EMBED_DIM), jnp.float32),
            pltpu.VMEM_SHARED((1024, _EMBED_DIM), jnp.float32),
        ] + [pltpu.SemaphoreType.DMA] * 13,
    )
    return f(seq, tok_table, pos_table)


def kernel(seq, tok_table, pos_table):
    return _embed(seq, tok_table, pos_table)


# speedup vs baseline: 1.0057x; 1.0057x over previous
import jax
import jax.numpy as jnp
from jax import lax
from jax.experimental import pallas as pl
from jax.experimental.pallas import tpu as pltpu
from jax.experimental.pallas import tpu_sc as plsc

_VOCAB = 100000
_MAX_LEN = 2048
_EMBED_DIM = 128
_BATCH = 4
_B = _BATCH * _MAX_LEN
_NC = 2
_NS = 16
_NW = _NC * _NS
_BPW = _B // _NW
_H = _BPW // 2


def _embed_body(seq_hbm, tok_hbm, pos_hbm, out_hbm,
                idx_v, buf0, buf1, sem_i, sem_p0, sem_p1, sem_g0, sem_g1,
                sem_s0, sem_s1):
    wid = lax.axis_index("s") * _NC + lax.axis_index("c")
    base = wid * _BPW
    b = base // _MAX_LEN
    col = lax.rem(base, _MAX_LEN)

    icopy = pltpu.async_copy(seq_hbm.at[b, pl.ds(col, _BPW)], idx_v, sem_i)
    p0 = pltpu.async_copy(pos_hbm.at[pl.ds(col, _H)], buf0, sem_p0)
    p1 = pltpu.async_copy(pos_hbm.at[pl.ds(col + _H, _H)], buf1, sem_p1)

    icopy.wait()
    p0.wait()
    g0 = pltpu.async_copy(tok_hbm.at[idx_v.at[pl.ds(0, _H)]], buf0, sem_g0,
                          add=True)
    p1.wait()
    g1 = pltpu.async_copy(tok_hbm.at[idx_v.at[pl.ds(_H, _H)]], buf1, sem_g1,
                          add=True)
    g0.wait()
    s0 = pltpu.async_copy(buf0, out_hbm.at[b, pl.ds(col, _H)], sem_s0)
    g1.wait()
    s1 = pltpu.async_copy(buf1, out_hbm.at[b, pl.ds(col + _H, _H)], sem_s1)
    s0.wait()
    s1.wait()


@jax.jit
def _embed(seq, tok_table, pos_table):
    mesh = plsc.VectorSubcoreMesh(core_axis_name="c", subcore_axis_name="s")
    f = pl.kernel(
        _embed_body,
        mesh=mesh,
        out_type=jax.ShapeDtypeStruct((_BATCH, _MAX_LEN, _EMBED_DIM),
                                      jnp.float32),
        scratch_types=[
            pltpu.VMEM((_BPW,), jnp.int32),
            pltpu.VMEM((_H, _EMBED_DIM), jnp.float32),
            pltpu.VMEM((_H, _EMBED_DIM), jnp.float32),
            pltpu.SemaphoreType.DMA,
            pltpu.SemaphoreType.DMA,
            pltpu.SemaphoreType.DMA,
            pltpu.SemaphoreType.DMA,
            pltpu.SemaphoreType.DMA,
            pltpu.SemaphoreType.DMA,
            pltpu.SemaphoreType.DMA,
        ],
    )
    return f(seq, tok_table, pos_table)


def kernel(seq, tok_table, pos_table):
    return _embed(seq, tok_table, pos_table)


# speedup vs baseline: 1.0087x; 1.0029x over previous
import jax
import jax.numpy as jnp
from jax import lax
from jax.experimental import pallas as pl
from jax.experimental.pallas import tpu as pltpu
from jax.experimental.pallas import tpu_sc as plsc

_VOCAB = 100000
_MAX_LEN = 2048
_EMBED_DIM = 128
_BATCH = 4
_B = _BATCH * _MAX_LEN
_NC = 2
_NS = 16
_NW = _NC * _NS
_BPW = _B // _NW
_H = _BPW // 2


def _embed_body(seq_hbm, tok_hbm, pos_hbm, out_hbm,
                idx_v, buf0, buf1, sem_i, sem_p0, sem_p1, sem_g0, sem_g1,
                sem_s0, sem_s1):
    wid = lax.axis_index("s") * _NC + lax.axis_index("c")
    base = wid * _BPW
    b = base // _MAX_LEN
    col = lax.rem(base, _MAX_LEN)

    icopy = pltpu.async_copy(seq_hbm.at[b, pl.ds(col, _BPW)], idx_v, sem_i)
    p0 = pltpu.async_copy(pos_hbm.at[pl.ds(col, _H)], buf0, sem_p0)
    p1 = pltpu.async_copy(pos_hbm.at[pl.ds(col + _H, _H)], buf1, sem_p1)

    icopy.wait()
    p0.wait()
    g0 = pltpu.async_copy(tok_hbm.at[idx_v.at[pl.ds(0, _H)]], buf0, sem_g0,
                          add=True)
    p1.wait()
    g1 = pltpu.async_copy(tok_hbm.at[idx_v.at[pl.ds(_H, _H)]], buf1, sem_g1,
                          add=True)
    g0.wait()
    s0 = pltpu.async_copy(buf0, out_hbm.at[b, pl.ds(col, _H)], sem_s0)
    g1.wait()
    s1 = pltpu.async_copy(buf1, out_hbm.at[b, pl.ds(col + _H, _H)], sem_s1)
    s0.wait()
    s1.wait()


@jax.jit
def _embed(seq, tok_table, pos_table):
    mesh = plsc.VectorSubcoreMesh(core_axis_name="c", subcore_axis_name="s")
    f = pl.kernel(
        _embed_body,
        mesh=mesh,
        out_type=jax.ShapeDtypeStruct((_BATCH, _MAX_LEN, _EMBED_DIM),
                                      jnp.float32),
        scratch_types=[
            pltpu.VMEM((_BPW,), jnp.int32),
            pltpu.VMEM((_H, _EMBED_DIM), jnp.float32),
            pltpu.VMEM((_H, _EMBED_DIM), jnp.float32),
            pltpu.SemaphoreType.DMA,
            pltpu.SemaphoreType.DMA,
            pltpu.SemaphoreType.DMA,
            pltpu.SemaphoreType.DMA,
            pltpu.SemaphoreType.DMA,
            pltpu.SemaphoreType.DMA,
            pltpu.SemaphoreType.DMA,
        ],
    )
    return f(seq, tok_table, pos_table)


def kernel(seq, tok_table, pos_table):
    return _embed(seq.astype(jnp.int32), tok_table, pos_table)
